# baseline (speedup 1.0000x reference)
import functools

import jax
import jax.numpy as jnp
from jax import lax
from jax.experimental import pallas as pl
from jax.experimental.pallas import tpu as pltpu
from jax.experimental.pallas import tpu_sc as plsc

_EPS = 1e-12
_LANES = 128
_SUB = 8
_NW = 32
_REP = 8
_CHUNK = 64


def _prep_body(lab_ref, emb_ref, w_ref, b_ref, st_ref, pat_ref):
    tab = emb_ref[...]
    mean = jnp.mean(tab, axis=-1, keepdims=True)
    var = jnp.mean(jnp.square(tab - mean), axis=-1, keepdims=True)
    nt = (tab - mean) / jnp.sqrt(var + _EPS) * w_ref[...][None, :] + b_ref[...][None, :]

    d = tab.shape[1]
    ntp = jnp.concatenate([nt, jnp.zeros((2, _LANES - d), jnp.float32)], axis=1)
    p_io = lax.broadcasted_iota(jnp.int32, st_ref.shape, 0) & 255
    s_io = lax.broadcasted_iota(jnp.int32, st_ref.shape, 1)
    bit = (p_io >> s_io) & 1
    st_ref[...] = jnp.where(bit == 1, ntp[1][None, None, :], ntp[0][None, None, :])

    labf = lab_ref[...].astype(jnp.float32)
    s_dim = labf.shape[1]
    d_io = lax.broadcasted_iota(jnp.int32, (s_dim, _LANES), 0)
    c_io = lax.broadcasted_iota(jnp.int32, (s_dim, _LANES), 1)
    w_bits = jnp.where((d_io >> 3) == c_io, 1 << (d_io & 7), 0).astype(jnp.float32)
    pat = jnp.dot(labf, w_bits, preferred_element_type=jnp.float32)
    pat_ref[...] = pat.astype(jnp.int32)


def _make_sc_kernel(n, s, d):
    n_grp = s // _SUB
    n_tiles = n * n_grp
    tiles_per_w = n_tiles // _NW
    rows_per_w = n // _NW
    n_chunks = tiles_per_w // _CHUNK
    mesh = plsc.VectorSubcoreMesh(core_axis_name="c", subcore_axis_name="s")

    @functools.partial(
        pl.kernel,
        out_type=jax.ShapeDtypeStruct((n_tiles, _SUB, _LANES), jnp.float32),
        mesh=mesh,
        compiler_params=pltpu.CompilerParams(needs_layout_passes=False),
        scratch_types=[
            pltpu.VMEM((rows_per_w * _LANES,), jnp.int32),
            pltpu.VMEM((1, _CHUNK), jnp.int32),
            pltpu.VMEM((1, _CHUNK), jnp.int32),
            pltpu.VMEM((_CHUNK, _SUB, _LANES), jnp.float32),
            pltpu.SemaphoreType.DMA,
            pltpu.SemaphoreType.DMA,
        ],
    )
    def sc_kernel(st_hbm, pat_hbm, out_hbm, patvm, tidx_v, oidx_v, buf, gsem, osem):
        cid = lax.axis_index("c")
        sid = lax.axis_index("s")
        wid = sid * 2 + cid
        rep_base = (wid & (_REP - 1)) * 256

        def stage(r, _):
            pltpu.sync_copy(pat_hbm.at[wid * rows_per_w + r],
                            patvm.at[pl.ds(r * _LANES, _LANES)])
            return ()

        lax.fori_loop(0, rows_per_w, stage, ())

        def body(c, _):
            g0 = c * _CHUNK
            for m in range(_CHUNK // 16):
                gv = lax.iota(jnp.int32, 16) + (g0 + m * 16)
                i_loc = gv // n_grp
                tj = gv % n_grp
                pq = plsc.load_gather(patvm, [i_loc * _LANES + tj])
                tidx_v[0, pl.ds(m * 16, 16)] = pq + rep_base
                oidx_v[0, pl.ds(m * 16, 16)] = gv + wid * tiles_per_w
            pltpu.async_copy(st_hbm.at[tidx_v.at[0]], buf, gsem).wait()
            pltpu.async_copy(buf, out_hbm.at[oidx_v.at[0]], osem).wait()
            return ()

        lax.fori_loop(0, n_chunks, body, ())

    return sc_kernel


def kernel(label_input, word_embeddings, ln_weight, ln_bias):
    n, s = label_input.shape
    _, d = word_embeddings.shape
    st, pat = pl.pallas_call(
        _prep_body,
        out_shape=[
            jax.ShapeDtypeStruct((_REP * 256, _SUB, _LANES), jnp.float32),
            jax.ShapeDtypeStruct((n, _LANES), jnp.int32),
        ],
    )(label_input, word_embeddings, ln_weight, ln_bias)
    out = _make_sc_kernel(n, s, d)(st, pat)
    return out.reshape(n, s, _LANES)[:, :, :d]


# speedup vs baseline: 1.1154x; 1.1154x over previous
import functools

import jax
import jax.numpy as jnp
from jax import lax
from jax.experimental import pallas as pl
from jax.experimental.pallas import tpu as pltpu
from jax.experimental.pallas import tpu_sc as plsc

_EPS = 1e-12
_LANES = 128
_SUB = 8
_NW = 32
_REP = 8
_CHUNK = 32


def _prep_body(lab_ref, emb_ref, w_ref, b_ref, st_ref, pat_ref):
    tab = emb_ref[...]
    mean = jnp.mean(tab, axis=-1, keepdims=True)
    var = jnp.mean(jnp.square(tab - mean), axis=-1, keepdims=True)
    nt = (tab - mean) / jnp.sqrt(var + _EPS) * w_ref[...][None, :] + b_ref[...][None, :]
    d = tab.shape[1]
    ntp = jnp.concatenate([nt, jnp.zeros((2, _LANES - d), jnp.float32)], axis=1)

    p_io = lax.broadcasted_iota(jnp.int32, st_ref.shape, 0) & 255
    s_io = lax.broadcasted_iota(jnp.int32, st_ref.shape, 1)
    bit = (p_io >> s_io) & 1
    st_ref[...] = jnp.where(bit == 1, ntp[1][None, None, :], ntp[0][None, None, :])

    labf = lab_ref[...].astype(jnp.float32)
    s_dim = labf.shape[1]
    d_io = lax.broadcasted_iota(jnp.int32, (s_dim, _LANES), 0)
    c_io = lax.broadcasted_iota(jnp.int32, (s_dim, _LANES), 1)
    w_bits = jnp.where((d_io >> 3) == c_io, 1 << (d_io & 7), 0).astype(jnp.float32)
    pat = jnp.dot(labf, w_bits, preferred_element_type=jnp.float32)
    pat_ref[...] = pat.astype(jnp.int32)


def _make_sc_kernel(n, s, d):
    n_grp = s // _SUB
    n_tiles = n * n_grp
    tiles_per_w = n_tiles // _NW
    rows_per_w = n // _NW
    n_chunks = tiles_per_w // _CHUNK
    mesh = plsc.VectorSubcoreMesh(core_axis_name="c", subcore_axis_name="s")

    @functools.partial(
        pl.kernel,
        out_type=jax.ShapeDtypeStruct((n_tiles, _SUB, _LANES), jnp.float32),
        mesh=mesh,
        compiler_params=pltpu.CompilerParams(needs_layout_passes=False),
        scratch_types=[
            pltpu.VMEM((rows_per_w, _LANES), jnp.int32),
            pltpu.VMEM((1, _LANES), jnp.int32),
            pltpu.VMEM((2, _CHUNK), jnp.int32),
            pltpu.VMEM((2, _CHUNK), jnp.int32),
            pltpu.VMEM((_CHUNK, _SUB, _LANES), jnp.float32),
            pltpu.VMEM((_CHUNK, _SUB, _LANES), jnp.float32),
            pltpu.SemaphoreType.DMA,
            pltpu.SemaphoreType.DMA,
            pltpu.SemaphoreType.DMA,
            pltpu.SemaphoreType.DMA,
        ],
    )
    def sc_kernel(st_hbm, pat_hbm, out_hbm, patvm, ridx_v, tidx_v, oidx_v,
                  buf_a, buf_b, gsem_a, gsem_b, osem_a, osem_b):
        cid = lax.axis_index("c")
        sid = lax.axis_index("s")
        wid = sid * 2 + cid
        rep_base = (wid & (_REP - 1)) * 256
        bufs = (buf_a, buf_b)
        gsems = (gsem_a, gsem_b)
        osems = (osem_a, osem_b)

        def gather_copy(p):
            return pltpu.make_async_copy(
                st_hbm.at[tidx_v.at[p]], bufs[p], gsems[p])

        def scatter_copy(p):
            return pltpu.make_async_copy(
                bufs[p], out_hbm.at[oidx_v.at[p]], osems[p])

        def compute_idx(c, p):
            for m in range(_CHUNK // 16):
                gv = lax.iota(jnp.int32, 16) + (c * _CHUNK + m * 16)
                i_loc = gv // n_grp
                tj = gv % n_grp
                pq = plsc.load_gather(patvm, [i_loc, tj])
                tidx_v[p, pl.ds(m * 16, 16)] = pq + rep_base
                oidx_v[p, pl.ds(m * 16, 16)] = gv + wid * tiles_per_w

        for m in range(rows_per_w // 16):
            ridx_v[0, pl.ds(m * 16, 16)] = (
                lax.iota(jnp.int32, 16) + (wid * rows_per_w + m * 16))
        pltpu.async_copy(pat_hbm.at[ridx_v.at[0]], patvm, gsem_a).wait()

        compute_idx(0, 0)
        gather_copy(0).start()
        compute_idx(1, 1)
        gather_copy(1).start()

        def body(k, _):
            c0 = 2 * k
            gather_copy(0).wait()
            scatter_copy(0).start()
            gather_copy(1).wait()
            scatter_copy(1).start()
            scatter_copy(0).wait()
            compute_idx(c0 + 2, 0)
            gather_copy(0).start()
            scatter_copy(1).wait()
            compute_idx(c0 + 3, 1)
            gather_copy(1).start()
            return ()

        lax.fori_loop(0, n_chunks // 2 - 1, body, ())

        gather_copy(0).wait()
        scatter_copy(0).start()
        gather_copy(1).wait()
        scatter_copy(1).start()
        scatter_copy(0).wait()
        scatter_copy(1).wait()

    return sc_kernel


def kernel(label_input, word_embeddings, ln_weight, ln_bias):
    n, s = label_input.shape
    _, d = word_embeddings.shape
    st, pat = pl.pallas_call(
        _prep_body,
        out_shape=[
            jax.ShapeDtypeStruct((_REP * 256, _SUB, _LANES), jnp.float32),
            jax.ShapeDtypeStruct((n, _LANES), jnp.int32),
        ],
    )(label_input, word_embeddings, ln_weight, ln_bias)
    out = _make_sc_kernel(n, s, d)(st, pat)
    return out.reshape(n, s, _LANES)[:, :, :d]


# speedup vs baseline: 1.2406x; 1.1123x over previous
import functools

import jax
import jax.numpy as jnp
from jax import lax
from jax.experimental import pallas as pl
from jax.experimental.pallas import tpu as pltpu
from jax.experimental.pallas import tpu_sc as plsc

_EPS = 1e-12
_LANES = 128
_SUB = 8
_NW = 32
_REP = 1
_CHUNK = 32


def _prep_body(lab_ref, emb_ref, w_ref, b_ref, st_ref, pat_ref):
    tab = emb_ref[...]
    mean = jnp.mean(tab, axis=-1, keepdims=True)
    var = jnp.mean(jnp.square(tab - mean), axis=-1, keepdims=True)
    nt = (tab - mean) / jnp.sqrt(var + _EPS) * w_ref[...][None, :] + b_ref[...][None, :]
    d = tab.shape[1]
    ntp = jnp.concatenate([nt, jnp.zeros((2, _LANES - d), jnp.float32)], axis=1)

    p_io = lax.broadcasted_iota(jnp.int32, st_ref.shape, 0) & 255
    s_io = lax.broadcasted_iota(jnp.int32, st_ref.shape, 1)
    bit = (p_io >> s_io) & 1
    st_ref[...] = jnp.where(bit == 1, ntp[1][None, None, :], ntp[0][None, None, :])

    labf = lab_ref[...].astype(jnp.float32)
    s_dim = labf.shape[1]
    d_io = lax.broadcasted_iota(jnp.int32, (s_dim, _LANES), 0)
    c_io = lax.broadcasted_iota(jnp.int32, (s_dim, _LANES), 1)
    w_bits = jnp.where((d_io >> 3) == c_io, 1 << (d_io & 7), 0).astype(jnp.float32)
    pat = jnp.dot(labf, w_bits, preferred_element_type=jnp.float32)
    pat_ref[...] = pat.astype(jnp.int32)


def _make_sc_kernel(n, s, d):
    n_grp = s // _SUB
    n_tiles = n * n_grp
    tiles_per_w = n_tiles // _NW
    rows_per_w = n // _NW
    n_chunks = tiles_per_w // _CHUNK
    mesh = plsc.VectorSubcoreMesh(core_axis_name="c", subcore_axis_name="s")

    @functools.partial(
        pl.kernel,
        out_type=jax.ShapeDtypeStruct((n_tiles, _SUB, _LANES), jnp.float32),
        mesh=mesh,
        compiler_params=pltpu.CompilerParams(needs_layout_passes=False),
        scratch_types=[
            pltpu.VMEM((rows_per_w, _LANES), jnp.int32),
            pltpu.VMEM((1, _LANES), jnp.int32),
            pltpu.VMEM((2, _CHUNK), jnp.int32),
            pltpu.VMEM((2, _CHUNK), jnp.int32),
            pltpu.VMEM((_CHUNK, _SUB, _LANES), jnp.float32),
            pltpu.VMEM((_CHUNK, _SUB, _LANES), jnp.float32),
            pltpu.VMEM_SHARED((256, _SUB, _LANES), jnp.float32),
            pltpu.SemaphoreType.DMA,
            pltpu.SemaphoreType.DMA,
            pltpu.SemaphoreType.DMA,
            pltpu.SemaphoreType.DMA,
        ],
    )
    def sc_kernel(st_hbm, pat_hbm, out_hbm, patvm, ridx_v, tidx_v, oidx_v,
                  buf_a, buf_b, st_sh, gsem_a, gsem_b, osem_a, osem_b):
        cid = lax.axis_index("c")
        sid = lax.axis_index("s")
        wid = sid * 2 + cid
        bufs = (buf_a, buf_b)
        gsems = (gsem_a, gsem_b)
        osems = (osem_a, osem_b)

        pltpu.sync_copy(st_hbm.at[pl.ds(sid * 16, 16)], buf_a.at[pl.ds(0, 16)])
        pltpu.sync_copy(buf_a.at[pl.ds(0, 16)], st_sh.at[pl.ds(sid * 16, 16)])
        plsc.subcore_barrier()

        def gather_copy(p):
            return pltpu.make_async_copy(
                st_sh.at[tidx_v.at[p]], bufs[p], gsems[p])

        def scatter_copy(p):
            return pltpu.make_async_copy(
                bufs[p], out_hbm.at[oidx_v.at[p]], osems[p])

        def compute_idx(c, p):
            for m in range(_CHUNK // 16):
                gv = lax.iota(jnp.int32, 16) + (c * _CHUNK + m * 16)
                i_loc = gv // n_grp
                tj = gv % n_grp
                pq = plsc.load_gather(patvm, [i_loc, tj])
                tidx_v[p, pl.ds(m * 16, 16)] = pq
                oidx_v[p, pl.ds(m * 16, 16)] = gv + wid * tiles_per_w

        for m in range(rows_per_w // 16):
            ridx_v[0, pl.ds(m * 16, 16)] = (
                lax.iota(jnp.int32, 16) + (wid * rows_per_w + m * 16))
        pltpu.async_copy(pat_hbm.at[ridx_v.at[0]], patvm, gsem_a).wait()

        compute_idx(0, 0)
        gather_copy(0).start()
        compute_idx(1, 1)
        gather_copy(1).start()

        def body(k, _):
            c0 = 2 * k
            gather_copy(0).wait()
            scatter_copy(0).start()
            gather_copy(1).wait()
            scatter_copy(1).start()
            scatter_copy(0).wait()
            compute_idx(c0 + 2, 0)
            gather_copy(0).start()
            scatter_copy(1).wait()
            compute_idx(c0 + 3, 1)
            gather_copy(1).start()
            return ()

        lax.fori_loop(0, n_chunks // 2 - 1, body, ())

        gather_copy(0).wait()
        scatter_copy(0).start()
        gather_copy(1).wait()
        scatter_copy(1).start()
        scatter_copy(0).wait()
        scatter_copy(1).wait()

    return sc_kernel


def kernel(label_input, word_embeddings, ln_weight, ln_bias):
    n, s = label_input.shape
    _, d = word_embeddings.shape
    st, pat = pl.pallas_call(
        _prep_body,
        out_shape=[
            jax.ShapeDtypeStruct((_REP * 256, _SUB, _LANES), jnp.float32),
            jax.ShapeDtypeStruct((n, _LANES), jnp.int32),
        ],
    )(label_input, word_embeddings, ln_weight, ln_bias)
    out = _make_sc_kernel(n, s, d)(st, pat)
    return out.reshape(n, s, _LANES)[:, :, :d]


# speedup vs baseline: 1.2423x; 1.0013x over previous
import functools

import jax
import jax.numpy as jnp
from jax import lax
from jax.experimental import pallas as pl
from jax.experimental.pallas import tpu as pltpu
from jax.experimental.pallas import tpu_sc as plsc

_EPS = 1e-12
_LANES = 128
_SUB = 8
_NW = 32
_REP = 1
_CHUNK = 40


def _prep_body(lab_ref, emb_ref, w_ref, b_ref, st_ref, pat_ref):
    tab = emb_ref[...]
    mean = jnp.mean(tab, axis=-1, keepdims=True)
    var = jnp.mean(jnp.square(tab - mean), axis=-1, keepdims=True)
    nt = (tab - mean) / jnp.sqrt(var + _EPS) * w_ref[...][None, :] + b_ref[...][None, :]
    d = tab.shape[1]
    ntp = jnp.concatenate([nt, jnp.zeros((2, _LANES - d), jnp.float32)], axis=1)

    p_io = lax.broadcasted_iota(jnp.int32, st_ref.shape, 0) & 255
    s_io = lax.broadcasted_iota(jnp.int32, st_ref.shape, 1)
    bit = (p_io >> s_io) & 1
    st_ref[...] = jnp.where(bit == 1, ntp[1][None, None, :], ntp[0][None, None, :])

    labf = lab_ref[...].astype(jnp.float32)
    s_dim = labf.shape[1]
    d_io = lax.broadcasted_iota(jnp.int32, (s_dim, _LANES), 0)
    c_io = lax.broadcasted_iota(jnp.int32, (s_dim, _LANES), 1)
    w_bits = jnp.where((d_io >> 3) == c_io, 1 << (d_io & 7), 0).astype(jnp.float32)
    pat = jnp.dot(labf, w_bits, preferred_element_type=jnp.float32)
    pat_ref[...] = pat.astype(jnp.int32)


def _make_sc_kernel(n, s, d):
    n_grp = s // _SUB
    n_tiles = n * n_grp
    tiles_per_w = n_tiles // _NW
    rows_per_w = n // _NW
    n_chunks = tiles_per_w // _CHUNK
    mesh = plsc.VectorSubcoreMesh(core_axis_name="c", subcore_axis_name="s")

    @functools.partial(
        pl.kernel,
        out_type=jax.ShapeDtypeStruct((n_tiles, _SUB, _LANES), jnp.float32),
        mesh=mesh,
        compiler_params=pltpu.CompilerParams(needs_layout_passes=False),
        scratch_types=[
            pltpu.VMEM((rows_per_w, _LANES), jnp.int32),
            pltpu.VMEM((1, _LANES), jnp.int32),
            pltpu.VMEM((2, 64), jnp.int32),
            pltpu.VMEM((_CHUNK, _SUB, _LANES), jnp.float32),
            pltpu.VMEM((_CHUNK, _SUB, _LANES), jnp.float32),
            pltpu.VMEM_SHARED((256, _SUB, _LANES), jnp.float32),
            pltpu.SemaphoreType.DMA,
            pltpu.SemaphoreType.DMA,
            pltpu.SemaphoreType.DMA,
            pltpu.SemaphoreType.DMA,
        ],
    )
    def sc_kernel(st_hbm, pat_hbm, out_hbm, patvm, ridx_v, tidx_v,
                  buf_a, buf_b, st_sh, gsem_a, gsem_b, osem_a, osem_b):
        cid = lax.axis_index("c")
        sid = lax.axis_index("s")
        wid = sid * 2 + cid
        bufs = (buf_a, buf_b)
        gsems = (gsem_a, gsem_b)
        osems = (osem_a, osem_b)

        pltpu.sync_copy(st_hbm.at[pl.ds(sid * 16, 16)], buf_a.at[pl.ds(0, 16)])
        pltpu.sync_copy(buf_a.at[pl.ds(0, 16)], st_sh.at[pl.ds(sid * 16, 16)])
        plsc.subcore_barrier()

        def gather_copy(p):
            return pltpu.make_async_copy(
                st_sh.at[tidx_v.at[p, pl.ds(0, _CHUNK)]], bufs[p], gsems[p])

        def scatter_copy(c, p):
            return pltpu.make_async_copy(
                bufs[p],
                out_hbm.at[pl.ds(wid * tiles_per_w + c * _CHUNK, _CHUNK)],
                osems[p])

        def compute_idx(c, p):
            for m in range((_CHUNK + 15) // 16):
                gv = lax.iota(jnp.int32, 16) + (c * _CHUNK + m * 16)
                gv = jnp.minimum(gv, tiles_per_w - 1)
                i_loc = gv // n_grp
                tj = gv % n_grp
                pq = plsc.load_gather(patvm, [i_loc, tj])
                tidx_v[p, pl.ds(m * 16, 16)] = pq

        for m in range(rows_per_w // 16):
            ridx_v[0, pl.ds(m * 16, 16)] = (
                lax.iota(jnp.int32, 16) + (wid * rows_per_w + m * 16))
        pltpu.async_copy(pat_hbm.at[ridx_v.at[0]], patvm, gsem_a).wait()

        compute_idx(0, 0)
        gather_copy(0).start()
        compute_idx(1, 1)
        gather_copy(1).start()

        def body(k, _):
            c0 = 2 * k
            gather_copy(0).wait()
            scatter_copy(c0, 0).start()
            gather_copy(1).wait()
            scatter_copy(c0 + 1, 1).start()
            scatter_copy(c0, 0).wait()
            compute_idx(c0 + 2, 0)
            gather_copy(0).start()
            scatter_copy(c0 + 1, 1).wait()
            compute_idx(c0 + 3, 1)
            gather_copy(1).start()
            return ()

        lax.fori_loop(0, n_chunks // 2 - 1, body, ())

        c_last = n_chunks - 2
        gather_copy(0).wait()
        scatter_copy(c_last, 0).start()
        gather_copy(1).wait()
        scatter_copy(c_last + 1, 1).start()
        scatter_copy(c_last, 0).wait()
        scatter_copy(c_last + 1, 1).wait()

    return sc_kernel


def kernel(label_input, word_embeddings, ln_weight, ln_bias):
    n, s = label_input.shape
    _, d = word_embeddings.shape
    st, pat = pl.pallas_call(
        _prep_body,
        out_shape=[
            jax.ShapeDtypeStruct((_REP * 256, _SUB, _LANES), jnp.float32),
            jax.ShapeDtypeStruct((n, _LANES), jnp.int32),
        ],
    )(label_input, word_embeddings, ln_weight, ln_bias)
    out = _make_sc_kernel(n, s, d)(st, pat)
    return out.reshape(n, s, _LANES)[:, :, :d]


# speedup vs baseline: 1.4527x; 1.1694x over previous
import functools

import jax
import jax.numpy as jnp
from jax import lax
from jax.experimental import pallas as pl
from jax.experimental.pallas import tpu as pltpu
from jax.experimental.pallas import tpu_sc as plsc

_EPS = 1e-12
_LANES = 128
_SUB = 8
_NW = 32
_CHUNK = 25
_DEPTH = 4


def _prep_body(lab_ref, emb_ref, w_ref, b_ref, st_ref, pat_ref):
    tab = emb_ref[...]
    mean = jnp.mean(tab, axis=-1, keepdims=True)
    var = jnp.mean(jnp.square(tab - mean), axis=-1, keepdims=True)
    nt = (tab - mean) / jnp.sqrt(var + _EPS) * w_ref[...][None, :] + b_ref[...][None, :]
    d = tab.shape[1]
    ntp = jnp.concatenate([nt, jnp.zeros((2, _LANES - d), jnp.float32)], axis=1)

    p_io = lax.broadcasted_iota(jnp.int32, st_ref.shape, 0) & 255
    s_io = lax.broadcasted_iota(jnp.int32, st_ref.shape, 1)
    bit = (p_io >> s_io) & 1
    st_ref[...] = jnp.where(bit == 1, ntp[1][None, None, :], ntp[0][None, None, :])

    labf = lab_ref[...].astype(jnp.float32)
    k_dim = labf.shape[1]
    d_io = lax.broadcasted_iota(jnp.int32, (k_dim, _LANES), 0)
    c_io = lax.broadcasted_iota(jnp.int32, (k_dim, _LANES), 1)
    w_bits = jnp.where((d_io >> 3) == c_io, 1 << (d_io & 7), 0).astype(jnp.float32)
    pat = jnp.dot(labf, w_bits, preferred_element_type=jnp.float32)
    pat_ref[...] = pat.astype(jnp.int32)


def _make_sc_kernel(n, s, d):
    n_grp = s // _SUB
    n_tiles = n * n_grp
    tiles_per_w = n_tiles // _NW
    prows_per_w = tiles_per_w // _LANES
    n_chunks = tiles_per_w // _CHUNK
    mesh = plsc.VectorSubcoreMesh(core_axis_name="c", subcore_axis_name="s")

    @functools.partial(
        pl.kernel,
        out_type=jax.ShapeDtypeStruct((n_tiles, _SUB, _LANES), jnp.float32),
        mesh=mesh,
        compiler_params=pltpu.CompilerParams(needs_layout_passes=False),
        scratch_types=[
            pltpu.VMEM((prows_per_w, _LANES), jnp.int32),
            pltpu.VMEM((1, 32), jnp.int32),
            pltpu.VMEM((_DEPTH, 32), jnp.int32),
            pltpu.VMEM((_CHUNK, _SUB, _LANES), jnp.float32),
            pltpu.VMEM((_CHUNK, _SUB, _LANES), jnp.float32),
            pltpu.VMEM((_CHUNK, _SUB, _LANES), jnp.float32),
            pltpu.VMEM((_CHUNK, _SUB, _LANES), jnp.float32),
            pltpu.VMEM_SHARED((256, _SUB, _LANES), jnp.float32),
            pltpu.SemaphoreType.DMA,
            pltpu.SemaphoreType.DMA,
            pltpu.SemaphoreType.DMA,
            pltpu.SemaphoreType.DMA,
            pltpu.SemaphoreType.DMA,
            pltpu.SemaphoreType.DMA,
            pltpu.SemaphoreType.DMA,
            pltpu.SemaphoreType.DMA,
        ],
    )
    def sc_kernel(st_hbm, pat_hbm, out_hbm, patvm, ridx_v, tidx_v,
                  buf0, buf1, buf2, buf3, st_sh,
                  g0, g1, g2, g3, o0, o1, o2, o3):
        cid = lax.axis_index("c")
        sid = lax.axis_index("s")
        wid = sid * 2 + cid
        bufs = (buf0, buf1, buf2, buf3)
        gsems = (g0, g1, g2, g3)
        osems = (o0, o1, o2, o3)

        pltpu.sync_copy(st_hbm.at[pl.ds(sid * 16, 16)], buf0.at[pl.ds(0, 16)])
        pltpu.sync_copy(buf0.at[pl.ds(0, 16)], st_sh.at[pl.ds(sid * 16, 16)])
        plsc.subcore_barrier()

        for m in range(2):
            ridx_v[0, pl.ds(m * 16, 16)] = jnp.minimum(
                lax.iota(jnp.int32, 16) + (wid * prows_per_w + m * 16),
                n_tiles // _LANES - 1)
        pltpu.async_copy(
            pat_hbm.at[ridx_v.at[0, pl.ds(0, prows_per_w)]], patvm, g0).wait()

        def compute_idx(c, p):
            for m in range(2):
                pos = lax.iota(jnp.int32, 16) + (c * _CHUNK + m * 16)
                pos = jnp.minimum(pos, tiles_per_w - 1)
                pq = plsc.load_gather(patvm, [pos >> 7, pos & 127])
                tidx_v[p, pl.ds(m * 16, 16)] = pq

        def gather_copy(p):
            return pltpu.make_async_copy(
                st_sh.at[tidx_v.at[p, pl.ds(0, _CHUNK)]], bufs[p], gsems[p])

        def scatter_copy(c, p):
            return pltpu.make_async_copy(
                bufs[p],
                out_hbm.at[pl.ds(wid * tiles_per_w + c * _CHUNK, _CHUNK)],
                osems[p])

        def step(c, p, wait_slot, do_gather, do_scatter):
            if wait_slot:
                scatter_copy(c - _DEPTH, p).wait()
            if do_gather:
                compute_idx(c, p)
                gather_copy(p).start()
            if do_scatter:
                p2 = (p + 2) % _DEPTH
                gather_copy(p2).wait()
                scatter_copy(c - 2, p2).start()

        step(0, 0, False, True, False)
        step(1, 1, False, True, False)
        step(2, 2, False, True, True)
        step(3, 3, False, True, True)

        def body(k, _):
            c0 = _DEPTH * k
            step(c0, 0, True, True, True)
            step(c0 + 1, 1, True, True, True)
            step(c0 + 2, 2, True, True, True)
            step(c0 + 3, 3, True, True, True)
            return ()

        lax.fori_loop(1, n_chunks // _DEPTH, body, ())

        step(n_chunks, 0, False, False, True)
        step(n_chunks + 1, 1, False, False, True)
        for c in range(n_chunks - _DEPTH, n_chunks):
            scatter_copy(c, c % _DEPTH).wait()

    return sc_kernel


def kernel(label_input, word_embeddings, ln_weight, ln_bias):
    n, s = label_input.shape
    _, d = word_embeddings.shape
    n_grp = s // _SUB
    lab2 = label_input.reshape(n * n_grp // _LANES, _SUB * _LANES)
    st, pat = pl.pallas_call(
        _prep_body,
        out_shape=[
            jax.ShapeDtypeStruct((256, _SUB, _LANES), jnp.float32),
            jax.ShapeDtypeStruct((n * n_grp // _LANES, _LANES), jnp.int32),
        ],
    )(lab2, word_embeddings, ln_weight, ln_bias)
    out = _make_sc_kernel(n, s, d)(st, pat)
    return out.reshape(n, s, _LANES)[:, :, :d]
